# baseline (speedup 1.0000x reference)
import functools

import jax
import jax.numpy as jnp
from jax import lax
from jax.experimental import pallas as pl
from jax.experimental.pallas import tpu as pltpu
from jax.experimental.pallas import tpu_sc as plsc

NC = 2
NS = 16
L = 16
NW = NC * NS

KNOTS = 16384
G = 16384
XSN = 24592
YSN = 16400
NQ = 4096 * 2048
QPW = NQ // NW
CHUNK = 16384
NCHUNK = QPW // CHUNK
U = 8
BPT = G // NS

_mesh = plsc.VectorSubcoreMesh(core_axis_name="c", subcore_axis_name="s")


@functools.partial(
    pl.kernel,
    out_type=jax.ShapeDtypeStruct((NQ,), jnp.float32),
    mesh=_mesh,
    compiler_params=pltpu.CompilerParams(needs_layout_passes=False),
    scratch_types=[
        pltpu.VMEM((XSN,), jnp.float32),
        pltpu.VMEM((YSN,), jnp.float32),
        pltpu.VMEM((G,), jnp.int32),
        pltpu.VMEM((CHUNK,), jnp.float32),
        pltpu.VMEM((CHUNK,), jnp.float32),
        pltpu.VMEM_SHARED((G,), jnp.int32),
    ],
)
def _spline_sc(xs_hbm, ys_hbm, q_hbm, out_hbm, xs_v, ys_v, b_v, q_v, o_v, b_sh):
    wid = lax.axis_index("c") * NS + lax.axis_index("s")
    sid = lax.axis_index("s")
    pltpu.sync_copy(xs_hbm, xs_v)
    pltpu.sync_copy(ys_hbm, ys_v)

    cbase = sid * BPT
    iota = lax.iota(jnp.int32, L)

    def bb_body(v, carry):
        off = cbase + v * (L * 4)
        for u in range(4):
            c0 = off + u * L
            cvec = c0 + iota
            i = jnp.zeros((L,), jnp.int32)
            for k in range(14, -1, -1):
                cand = i + (1 << k)
                xm = plsc.load_gather(xs_v, [cand])
                kb = jnp.where(xm >= jnp.float32(1.0), jnp.int32(G),
                               jnp.minimum(
                                   (xm * jnp.float32(G)).astype(jnp.int32),
                                   jnp.int32(G - 1)))
                i = jnp.where(kb < cvec, cand, i)
            b_v[pl.ds(c0, L)] = i
        return carry

    lax.fori_loop(0, BPT // (L * 4), bb_body, jnp.int32(0), unroll=1)
    pltpu.sync_copy(b_v.at[pl.ds(cbase, BPT)], b_sh.at[pl.ds(cbase, BPT)])
    plsc.subcore_barrier()
    pltpu.sync_copy(b_sh, b_v)

    base_w = wid * QPW

    def chunk_body(cc, carry):
        base = base_w + cc * CHUNK
        pltpu.sync_copy(q_hbm.at[pl.ds(base, CHUNK)], q_v)

        def body(it):
            off = it * (L * U)
            qs, idx, xrs = [], [], []
            for u in range(U):
                q = q_v[pl.ds(off + u * L, L)]
                c = jnp.minimum((q * jnp.float32(G)).astype(jnp.int32),
                                jnp.int32(G - 1))
                i = plsc.load_gather(b_v, [c])
                for s in (8, 4, 2, 1):
                    cand = i + s
                    xm = plsc.load_gather(xs_v, [cand])
                    i = jnp.where(xm <= q, cand, i)
                qs.append(q)
                idx.append(i)
                xrs.append(plsc.load_gather(xs_v, [i + 1]))

            def more(st):
                need = [st[1][u] <= qs[u] for u in range(U)]
                m = need[0]
                for u in range(1, U):
                    m = m | need[u]
                return jnp.any(m)

            def bump(st):
                ii, xx = st
                ni, nx = [], []
                for u in range(U):
                    i2 = jnp.where(xx[u] <= qs[u], ii[u] + 1, ii[u])
                    ni.append(i2)
                    nx.append(plsc.load_gather(xs_v, [i2 + 1]))
                return tuple(ni), tuple(nx)

            idx, xrs = lax.while_loop(more, bump, (tuple(idx), tuple(xrs)))

            for u in range(U):
                q, i, xr = qs[u], idx[u], xrs[u]
                xl = plsc.load_gather(xs_v, [i])
                yl = plsc.load_gather(ys_v, [i])
                yr = plsc.load_gather(ys_v, [i + 1])
                eq = xl == xr
                denom = jnp.where(eq, jnp.float32(1.0), xr - xl)
                w = jnp.where(eq, jnp.float32(0.0), (q - xl) / denom)
                o_v[pl.ds(off + u * L, L)] = yl + w * (yr - yl)

        plsc.parallel_loop(0, CHUNK // (L * U))(body)
        pltpu.sync_copy(o_v, out_hbm.at[pl.ds(base, CHUNK)])
        return carry

    lax.fori_loop(0, NCHUNK, chunk_body, jnp.int32(0), unroll=1)


def kernel(x, y, x_new):
    order = jnp.argsort(x)
    xs = x[order]
    ys = y[order]
    xsearch = jnp.concatenate(
        [xs[:1], xs, jnp.full((XSN - KNOTS - 1,), jnp.inf, jnp.float32)])
    ys_pad = jnp.concatenate(
        [ys[:1], ys, ys[-1:], jnp.zeros((YSN - KNOTS - 2,), jnp.float32)])
    out = _spline_sc(xsearch, ys_pad, x_new.reshape(-1))
    return out.reshape(x_new.shape)


# speedup vs baseline: 1.8570x; 1.8570x over previous
import functools

import jax
import jax.numpy as jnp
from jax import lax
from jax.experimental import pallas as pl
from jax.experimental.pallas import tpu as pltpu
from jax.experimental.pallas import tpu_sc as plsc

NC = 2
NS = 16
L = 16
NW = NC * NS

KNOTS = 16384
G = 16384
XSN = 24592
YSN = 16400
NQ = 4096 * 2048
QPW = NQ // NW
CHUNK = 16384
NCHUNK = QPW // CHUNK
U = 16
BPT = G // NS

_mesh = plsc.VectorSubcoreMesh(core_axis_name="c", subcore_axis_name="s")


@functools.partial(
    pl.kernel,
    out_type=jax.ShapeDtypeStruct((NQ,), jnp.float32),
    mesh=_mesh,
    compiler_params=pltpu.CompilerParams(needs_layout_passes=False),
    scratch_types=[
        pltpu.VMEM((XSN,), jnp.float32),
        pltpu.VMEM((YSN,), jnp.float32),
        pltpu.VMEM((G,), jnp.int32),
        pltpu.VMEM((CHUNK,), jnp.float32),
        pltpu.VMEM((CHUNK,), jnp.float32),
        pltpu.VMEM_SHARED((G,), jnp.int32),
    ],
)
def _spline_sc(xs_hbm, ys_hbm, q_hbm, out_hbm, xs_v, ys_v, b_v, q_v, o_v, b_sh):
    wid = lax.axis_index("c") * NS + lax.axis_index("s")
    sid = lax.axis_index("s")
    pltpu.sync_copy(xs_hbm, xs_v)
    pltpu.sync_copy(ys_hbm, ys_v)

    cbase = sid * BPT
    iota = lax.iota(jnp.int32, L)

    def bb_body(v, carry):
        off = cbase + v * (L * 4)
        for u in range(4):
            c0 = off + u * L
            cvec = c0 + iota
            i = jnp.zeros((L,), jnp.int32)
            for k in range(14, -1, -1):
                cand = i + (1 << k)
                xm = plsc.load_gather(xs_v, [cand])
                kb = jnp.where(xm >= jnp.float32(1.0), jnp.int32(G),
                               jnp.minimum(
                                   (xm * jnp.float32(G)).astype(jnp.int32),
                                   jnp.int32(G - 1)))
                i = jnp.where(kb < cvec, cand, i)
            b_v[pl.ds(c0, L)] = i
        return carry

    lax.fori_loop(0, BPT // (L * 4), bb_body, jnp.int32(0), unroll=1)
    pltpu.sync_copy(b_v.at[pl.ds(cbase, BPT)], b_sh.at[pl.ds(cbase, BPT)])
    plsc.subcore_barrier()
    pltpu.sync_copy(b_sh, b_v)

    base_w = wid * QPW

    def chunk_body(cc, carry):
        base = base_w + cc * CHUNK
        pltpu.sync_copy(q_hbm.at[pl.ds(base, CHUNK)], q_v)

        def body(it, miss):
            off = it * (L * U)
            for u in range(U):
                q = q_v[pl.ds(off + u * L, L)]
                c = jnp.minimum((q * jnp.float32(G)).astype(jnp.int32),
                                jnp.int32(G - 1))
                i = plsc.load_gather(b_v, [c])
                for s in (8, 4, 2, 1):
                    cand = i + s
                    xm = plsc.load_gather(xs_v, [cand])
                    i = jnp.where(xm <= q, cand, i)
                xr = plsc.load_gather(xs_v, [i + 1])
                miss = miss | (xr <= q)
                xl = plsc.load_gather(xs_v, [i])
                yl = plsc.load_gather(ys_v, [i])
                yr = plsc.load_gather(ys_v, [i + 1])
                eq = xl == xr
                denom = jnp.where(eq, jnp.float32(1.0), xr - xl)
                w = jnp.where(eq, jnp.float32(0.0), (q - xl) / denom)
                o_v[pl.ds(off + u * L, L)] = yl + w * (yr - yl)
            return miss

        miss0 = jnp.zeros((L,), jnp.bool_)
        miss = plsc.parallel_loop(0, CHUNK // (L * U), carry=miss0)(body)

        @pl.when(jnp.any(miss))
        def _redo():
            def rbody(it, carry3):
                off = it * L
                q = q_v[pl.ds(off, L)]
                i = jnp.zeros((L,), jnp.int32)
                for k in range(14, -1, -1):
                    cand = i + (1 << k)
                    xm = plsc.load_gather(xs_v, [cand])
                    i = jnp.where(xm <= q, cand, i)
                xl = plsc.load_gather(xs_v, [i])
                xr = plsc.load_gather(xs_v, [i + 1])
                yl = plsc.load_gather(ys_v, [i])
                yr = plsc.load_gather(ys_v, [i + 1])
                eq = xl == xr
                denom = jnp.where(eq, jnp.float32(1.0), xr - xl)
                w = jnp.where(eq, jnp.float32(0.0), (q - xl) / denom)
                o_v[pl.ds(off, L)] = yl + w * (yr - yl)
                return carry3

            lax.fori_loop(0, CHUNK // L, rbody, jnp.int32(0), unroll=1)
        pltpu.sync_copy(o_v, out_hbm.at[pl.ds(base, CHUNK)])
        return carry

    lax.fori_loop(0, NCHUNK, chunk_body, jnp.int32(0), unroll=1)


def kernel(x, y, x_new):
    order = jnp.argsort(x)
    xs = x[order]
    ys = y[order]
    xsearch = jnp.concatenate(
        [xs[:1], xs, jnp.full((XSN - KNOTS - 1,), jnp.inf, jnp.float32)])
    ys_pad = jnp.concatenate(
        [ys[:1], ys, ys[-1:], jnp.zeros((YSN - KNOTS - 2,), jnp.float32)])
    out = _spline_sc(xsearch, ys_pad, x_new.reshape(-1))
    return out.reshape(x_new.shape)
